# Initial kernel scaffold; baseline (speedup 1.0000x reference)
import jax
import jax.numpy as jnp
from jax.experimental import pallas as pl


def kernel(x, message_edge_index, target_edge_index, target_edge_weights, W_rel, b_rel, W_root):
    raise NotImplementedError("write your pallas kernel here")


# speedup vs baseline: 5.2562x; 5.2562x over previous
import functools

import jax
import jax.numpy as jnp
from jax import lax
from jax.experimental import pallas as pl
from jax.experimental.pallas import tpu as pltpu
from jax.experimental.pallas import tpu_sc as plsc

N_NODES = 10000
D = 128
E_MSG = 256000
E_TGT = 64000

NC = 2
NS = 16
NW = NC * NS

CHUNK = 80
MSG_PER_W = E_MSG // NW
TGT_PER_W = E_TGT // NW
N_PAD = 10240
ROWS_PER_TILE = N_PAD // NS
ZCH = 128
CNT_W = 16


def _sc_aggregate(x, msg_src, msg_dst, tgt_src, tgt_dst, tgt_w16):
    mesh = plsc.VectorSubcoreMesh(core_axis_name="c", subcore_axis_name="s")

    @functools.partial(
        pl.kernel,
        mesh=mesh,
        out_type=[
            jax.ShapeDtypeStruct((NC, N_PAD, D), jnp.float32),
            jax.ShapeDtypeStruct((NC, N_PAD, CNT_W), jnp.float32),
        ],
        scratch_types=[
            pltpu.VMEM((CHUNK,), jnp.int32),
            pltpu.VMEM((CHUNK,), jnp.int32),
            pltpu.VMEM((CHUNK, D), jnp.float32),
            pltpu.VMEM((CHUNK, CNT_W), jnp.float32),
            pltpu.VMEM((CHUNK, CNT_W), jnp.float32),
            pltpu.VMEM_SHARED((N_PAD, D), jnp.float32),
            pltpu.VMEM_SHARED((N_PAD, CNT_W), jnp.float32),
            pltpu.SemaphoreType.DMA,
        ],
    )
    def k(x_hbm, ms_hbm, md_hbm, ts_hbm, td_hbm, tw_hbm,
          sum_out, cnt_out,
          sidx, didx, rows, ones, wbuf, acc, cnt, sem):
        cid = lax.axis_index("c")
        sid = lax.axis_index("s")
        wid = cid * NS + sid

        def fill_rows_zero(r, carry):
            for j in range(D // 16):
                rows[r, pl.ds(j * 16, 16)] = jnp.zeros((16,), jnp.float32)
            ones[r, pl.ds(0, 16)] = jnp.zeros((16,), jnp.float32)
            return carry
        lax.fori_loop(0, CHUNK, fill_rows_zero, 0)

        base_row = sid * ROWS_PER_TILE
        for t in range(ROWS_PER_TILE // CHUNK):
            r0 = base_row + t * CHUNK
            for g in range(CHUNK // 16):
                didx[pl.ds(g * 16, 16)] = (
                    lax.iota(jnp.int32, 16) + (r0 + g * 16))
            pltpu.sync_copy(rows, acc.at[didx])
            pltpu.sync_copy(ones, cnt.at[didx])

        def fill_ones(r, carry):
            ones[r, pl.ds(0, 16)] = jnp.ones((16,), jnp.float32)
            return carry
        lax.fori_loop(0, CHUNK, fill_ones, 0)
        plsc.subcore_barrier()

        mbase = wid * MSG_PER_W

        def mstep(i, carry):
            b = mbase + i * CHUNK
            pltpu.sync_copy(ms_hbm.at[pl.ds(b, CHUNK)], sidx)
            pltpu.sync_copy(md_hbm.at[pl.ds(b, CHUNK)], didx)
            pltpu.async_copy(x_hbm.at[sidx], rows, sem).wait()
            pltpu.sync_copy(rows, acc.at[didx], add=True)
            pltpu.sync_copy(ones, cnt.at[didx], add=True)
            return carry
        lax.fori_loop(0, MSG_PER_W // CHUNK, mstep, 0)

        tbase = wid * TGT_PER_W

        def tstep(i, carry):
            b = tbase + i * CHUNK
            pltpu.sync_copy(ts_hbm.at[pl.ds(b, CHUNK)], sidx)
            pltpu.sync_copy(td_hbm.at[pl.ds(b, CHUNK)], didx)
            pltpu.sync_copy(tw_hbm.at[pl.ds(b, CHUNK)], wbuf)
            pltpu.async_copy(x_hbm.at[sidx], rows, sem).wait()

            def scale_row(r, c2):
                ws = wbuf[r, pl.ds(0, 16)]
                for j in range(D // 16):
                    rows[r, pl.ds(j * 16, 16)] = rows[r, pl.ds(j * 16, 16)] * ws
                return c2
            lax.fori_loop(0, CHUNK, scale_row, 0)

            pltpu.sync_copy(rows, acc.at[didx], add=True)
            pltpu.sync_copy(ones, cnt.at[didx], add=True)
            return carry
        lax.fori_loop(0, TGT_PER_W // CHUNK, tstep, 0)

        plsc.subcore_barrier()

        for t in range(ROWS_PER_TILE // CHUNK):
            r0 = base_row + t * CHUNK
            for g in range(CHUNK // 16):
                didx[pl.ds(g * 16, 16)] = (
                    lax.iota(jnp.int32, 16) + (r0 + g * 16))
            pltpu.async_copy(acc.at[didx], rows, sem).wait()
            pltpu.sync_copy(rows, sum_out.at[cid, pl.ds(r0, CHUNK)])
            pltpu.async_copy(cnt.at[didx], ones, sem).wait()
            pltpu.sync_copy(ones, cnt_out.at[cid, pl.ds(r0, CHUNK)])

    return k(x, msg_src, msg_dst, tgt_src, tgt_dst, tgt_w16)


BLK = 1000


def _tc_body(sum_ref, cnt_ref, x_ref, wrel_ref, brel_ref, wroot_ref, o_ref):
    s = sum_ref[0] + sum_ref[1]
    c = cnt_ref[0][:, 0:1] + cnt_ref[1][:, 0:1]
    mean = s / jnp.clip(c, 1.0, None)
    o_ref[...] = (
        jnp.dot(mean, wrel_ref[...], preferred_element_type=jnp.float32)
        + jnp.dot(x_ref[...], wroot_ref[...], preferred_element_type=jnp.float32)
        + brel_ref[...]
    )


def _tc_combine(sums, cnts, x, W_rel, b_rel, W_root):
    grid = (N_NODES // BLK,)
    return pl.pallas_call(
        _tc_body,
        grid=grid,
        in_specs=[
            pl.BlockSpec((NC, BLK, D), lambda i: (0, i, 0)),
            pl.BlockSpec((NC, BLK, CNT_W), lambda i: (0, i, 0)),
            pl.BlockSpec((BLK, D), lambda i: (i, 0)),
            pl.BlockSpec((D, D), lambda i: (0, 0)),
            pl.BlockSpec((1, D), lambda i: (0, 0)),
            pl.BlockSpec((D, D), lambda i: (0, 0)),
        ],
        out_specs=pl.BlockSpec((BLK, D), lambda i: (i, 0)),
        out_shape=jax.ShapeDtypeStruct((N_NODES, D), jnp.float32),
    )(sums, cnts, x, W_rel, b_rel, W_root)


def kernel(x, message_edge_index, target_edge_index, target_edge_weights,
           W_rel, b_rel, W_root):
    msg_src = message_edge_index[0]
    msg_dst = message_edge_index[1]
    tgt_src = target_edge_index[0]
    tgt_dst = target_edge_index[1]

    tgt_w16 = jnp.broadcast_to(target_edge_weights[:, None], (E_TGT, CNT_W))
    sums, cnts = _sc_aggregate(x, msg_src, msg_dst, tgt_src, tgt_dst, tgt_w16)
    out = _tc_combine(sums, cnts, x, W_rel, b_rel.reshape(1, D), W_root)
    return (out, target_edge_weights)
